# baseline (speedup 1.0000x reference)
import functools

import jax
import jax.numpy as jnp
from jax import lax
from jax.experimental import pallas as pl
from jax.experimental.pallas import tpu as pltpu

TOP_K = 8
N_EXPERTS = 64
N_GROUP = 8
GROUP_SIZE = N_EXPERTS // N_GROUP
TOPK_GROUP = 4
SCALING = 1.0
HIDDEN = 4096

_NEG = -1e30


def _router_body(x_ref, w_ref, b_ref, o_ref, *, block_rows):
    xb = x_ref[...]
    logits = jnp.dot(xb, w_ref[...], preferred_element_type=jnp.float32)
    scores = jax.nn.sigmoid(logits)
    s4c = scores + b_ref[...]

    R = block_rows
    i8 = lax.broadcasted_iota(jnp.int32, (R, GROUP_SIZE), 1)

    gsums = []
    for g in range(N_GROUP):
        sl = s4c[:, g * GROUP_SIZE:(g + 1) * GROUP_SIZE]
        m1 = jnp.max(sl, axis=1, keepdims=True)
        first = jnp.min(
            jnp.where(sl == m1, i8, GROUP_SIZE), axis=1, keepdims=True)
        m2 = jnp.max(jnp.where(i8 == first, _NEG, sl), axis=1, keepdims=True)
        gsums.append(m1 + m2)
    gsum = jnp.concatenate(gsums, axis=1)

    work = gsum
    gmask = jnp.zeros((R, N_GROUP), jnp.float32)
    for _ in range(TOPK_GROUP):
        m = jnp.max(work, axis=1, keepdims=True)
        first = jnp.min(
            jnp.where(work == m, i8, N_GROUP), axis=1, keepdims=True)
        sel = i8 == first
        gmask = jnp.where(sel, 1.0, gmask)
        work = jnp.where(sel, _NEG, work)

    gi = lax.broadcasted_iota(jnp.int32, (N_GROUP, N_EXPERTS), 0)
    li = lax.broadcasted_iota(jnp.int32, (N_GROUP, N_EXPERTS), 1)
    memb = (li // GROUP_SIZE == gi).astype(jnp.float32)
    emask = jnp.dot(gmask, memb, preferred_element_type=jnp.float32)
    ms = jnp.where(emask > 0.0, s4c, 0.0)

    l64 = lax.broadcasted_iota(jnp.int32, (R, N_EXPERTS), 1)
    work = ms
    cols = []
    for _ in range(TOP_K):
        m = jnp.max(work, axis=1, keepdims=True)
        first = jnp.min(
            jnp.where(work == m, l64, N_EXPERTS), axis=1, keepdims=True)
        onehot = l64 == first
        cols.append(
            jnp.sum(jnp.where(onehot, scores, 0.0), axis=1, keepdims=True))
        work = jnp.where(onehot, _NEG, work)
    w_sel = jnp.concatenate(cols, axis=1)

    denom = jnp.sum(w_sel, axis=1, keepdims=True) + 1e-20
    o_ref[...] = (w_sel / denom) * SCALING


def kernel(hidden_states, kernel, e_score_correction_bias):
    x = hidden_states.reshape(-1, HIDDEN)
    rows = x.shape[0]
    block_rows = 512
    grid = (rows // block_rows,)
    bias2d = e_score_correction_bias.reshape(1, N_EXPERTS)

    return pl.pallas_call(
        functools.partial(_router_body, block_rows=block_rows),
        grid=grid,
        in_specs=[
            pl.BlockSpec((block_rows, HIDDEN), lambda i: (i, 0)),
            pl.BlockSpec((HIDDEN, N_EXPERTS), lambda i: (0, 0)),
            pl.BlockSpec((1, N_EXPERTS), lambda i: (0, 0)),
        ],
        out_specs=pl.BlockSpec((block_rows, TOP_K), lambda i: (i, 0)),
        out_shape=jax.ShapeDtypeStruct((rows, TOP_K), jnp.float32),
        compiler_params=pltpu.CompilerParams(
            dimension_semantics=("arbitrary",),
        ),
    )(x, kernel, bias2d)


# speedup vs baseline: 3.2440x; 3.2440x over previous
import functools

import jax
import jax.numpy as jnp
from jax import lax
from jax.experimental import pallas as pl
from jax.experimental.pallas import tpu as pltpu

TOP_K = 8
N_EXPERTS = 64
N_GROUP = 8
GROUP_SIZE = N_EXPERTS // N_GROUP
TOPK_GROUP = 4
SCALING = 1.0
HIDDEN = 4096

_NEG = -1e30


def _router_body(x_ref, w_ref, b_ref, o_ref, *, block_rows):
    R = block_rows
    xb = x_ref[...]
    logits = jnp.dot(xb, w_ref[...], preferred_element_type=jnp.float32)
    lt = logits.T
    scores = jax.nn.sigmoid(lt)
    s4c = scores + b_ref[...]

    s3 = s4c.reshape(N_GROUP, GROUP_SIZE, R)
    i8 = lax.broadcasted_iota(jnp.int32, (N_GROUP, GROUP_SIZE, R), 1)

    m1 = jnp.max(s3, axis=1)
    first = jnp.min(
        jnp.where(s3 == m1[:, None, :], i8, GROUP_SIZE), axis=1)
    m2 = jnp.max(
        jnp.where(i8 == first[:, None, :], _NEG, s3), axis=1)
    gsum = m1 + m2

    ig = lax.broadcasted_iota(jnp.int32, (N_GROUP, R), 0)
    work = gsum
    gmask = jnp.zeros((N_GROUP, R), jnp.float32)
    for _ in range(TOPK_GROUP):
        m = jnp.max(work, axis=0, keepdims=True)
        first = jnp.min(
            jnp.where(work == m, ig, N_GROUP), axis=0, keepdims=True)
        sel = ig == first
        gmask = jnp.where(sel, 1.0, gmask)
        work = jnp.where(sel, _NEG, work)

    ms3 = jnp.where(gmask[:, None, :] > 0.0, s3, 0.0)

    le = lax.broadcasted_iota(jnp.int32, (N_GROUP, GROUP_SIZE, R), 0) * \
        GROUP_SIZE + i8
    sc3 = scores.reshape(N_GROUP, GROUP_SIZE, R)
    work3 = ms3
    cols = []
    for _ in range(TOP_K):
        m = jnp.max(work3, axis=(0, 1))
        first = jnp.min(
            jnp.where(work3 == m[None, None, :], le, N_EXPERTS), axis=(0, 1))
        oh = le == first[None, None, :]
        cols.append(jnp.sum(jnp.where(oh, sc3, 0.0), axis=(0, 1))[None, :])
        work3 = jnp.where(oh, _NEG, work3)
    w_sel = jnp.concatenate(cols, axis=0)

    denom = jnp.sum(w_sel, axis=0, keepdims=True) + 1e-20
    o_ref[...] = ((w_sel / denom) * SCALING).T


def kernel(hidden_states, kernel, e_score_correction_bias):
    x = hidden_states.reshape(-1, HIDDEN)
    rows = x.shape[0]
    block_rows = 512
    grid = (rows // block_rows,)
    bias_col = e_score_correction_bias.reshape(N_EXPERTS, 1)

    return pl.pallas_call(
        functools.partial(_router_body, block_rows=block_rows),
        grid=grid,
        in_specs=[
            pl.BlockSpec((block_rows, HIDDEN), lambda i: (i, 0)),
            pl.BlockSpec((HIDDEN, N_EXPERTS), lambda i: (0, 0)),
            pl.BlockSpec((N_EXPERTS, 1), lambda i: (0, 0)),
        ],
        out_specs=pl.BlockSpec((block_rows, TOP_K), lambda i: (i, 0)),
        out_shape=jax.ShapeDtypeStruct((rows, TOP_K), jnp.float32),
        compiler_params=pltpu.CompilerParams(
            dimension_semantics=("arbitrary",),
        ),
    )(x, kernel, bias_col)


# speedup vs baseline: 3.4200x; 1.0543x over previous
import functools

import jax
import jax.numpy as jnp
from jax import lax
from jax.experimental import pallas as pl
from jax.experimental.pallas import tpu as pltpu

TOP_K = 8
N_EXPERTS = 64
N_GROUP = 8
GROUP_SIZE = N_EXPERTS // N_GROUP
TOPK_GROUP = 4
SCALING = 1.0
HIDDEN = 4096

_NEG = -1e30


def _router_body(x_ref, w_ref, b_ref, o_ref, *, block_rows):
    R = block_rows
    xb = x_ref[...]
    logits = jnp.dot(xb, w_ref[...], preferred_element_type=jnp.float32)
    lt = logits.T
    scores = jax.nn.sigmoid(lt)
    s4c = scores + b_ref[...]

    s3 = s4c.reshape(N_GROUP, GROUP_SIZE, R)
    i8 = lax.broadcasted_iota(jnp.int32, (N_GROUP, GROUP_SIZE, R), 1)

    m1 = jnp.max(s3, axis=1)
    first = jnp.min(
        jnp.where(s3 == m1[:, None, :], i8, GROUP_SIZE), axis=1)
    m2 = jnp.max(
        jnp.where(i8 == first[:, None, :], _NEG, s3), axis=1)
    gsum = m1 + m2

    ig = lax.broadcasted_iota(jnp.int32, (N_GROUP, R), 0)
    work = gsum
    gmask = jnp.zeros((N_GROUP, R), jnp.float32)
    for _ in range(TOPK_GROUP):
        m = jnp.max(work, axis=0, keepdims=True)
        first = jnp.min(
            jnp.where(work == m, ig, N_GROUP), axis=0, keepdims=True)
        sel = ig == first
        gmask = jnp.where(sel, 1.0, gmask)
        work = jnp.where(sel, _NEG, work)

    ms3 = jnp.where(gmask[:, None, :] > 0.0, s3, 0.0)

    le = lax.broadcasted_iota(jnp.int32, (N_GROUP, GROUP_SIZE, R), 0) * \
        GROUP_SIZE + i8
    sc3 = scores.reshape(N_GROUP, GROUP_SIZE, R)
    work3 = ms3
    cols = []
    for _ in range(TOP_K):
        m = jnp.max(work3, axis=(0, 1))
        first = jnp.min(
            jnp.where(work3 == m[None, None, :], le, N_EXPERTS), axis=(0, 1))
        oh = le == first[None, None, :]
        cols.append(jnp.sum(jnp.where(oh, sc3, 0.0), axis=(0, 1))[None, :])
        work3 = jnp.where(oh, _NEG, work3)
    w_sel = jnp.concatenate(cols, axis=0)

    denom = jnp.sum(w_sel, axis=0, keepdims=True) + 1e-20
    o_ref[...] = ((w_sel / denom) * SCALING).T


def kernel(hidden_states, kernel, e_score_correction_bias):
    x = hidden_states.reshape(-1, HIDDEN)
    rows = x.shape[0]
    block_rows = 1024
    grid = (rows // block_rows,)
    bias_col = e_score_correction_bias.reshape(N_EXPERTS, 1)

    return pl.pallas_call(
        functools.partial(_router_body, block_rows=block_rows),
        grid=grid,
        in_specs=[
            pl.BlockSpec((block_rows, HIDDEN), lambda i: (i, 0)),
            pl.BlockSpec((HIDDEN, N_EXPERTS), lambda i: (0, 0)),
            pl.BlockSpec((N_EXPERTS, 1), lambda i: (0, 0)),
        ],
        out_specs=pl.BlockSpec((block_rows, TOP_K), lambda i: (i, 0)),
        out_shape=jax.ShapeDtypeStruct((rows, TOP_K), jnp.float32),
        compiler_params=pltpu.CompilerParams(
            dimension_semantics=("arbitrary",),
        ),
    )(x, kernel, bias_col)
